# baseline (speedup 1.0000x reference)
import jax
import jax.numpy as jnp
from jax import lax
from jax.experimental import pallas as pl
from jax.experimental.pallas import tpu as pltpu
from jax.experimental.pallas import tpu_sc as plsc

N_NODES = 10000
D = 256
DH = 128
NC = 2
NS = 16
B = 64
EPT = 10240
NBLK = EPT // B
E_PAD = NS * EPT
HEDGE = EPT // 2
HBLK = HEDGE // B
ACC_ROWS = 10112
ROWS_PER_TILE = ACC_ROWS // NS
DUMMY = N_NODES


def _sc_body(xcat, src_hbm, dst_hbm, h2, src_v, dst_v, ida, idb, hist,
             buf0, buf1, ssrc, sdst, acc, g0, g1, s0, s1):
    c = lax.axis_index("c")
    s = lax.axis_index("s")
    i32 = jnp.int32
    iota16 = lax.iota(i32, 16)
    ones16 = jnp.ones((16,), i32)
    z16 = jnp.zeros((16,), i32)

    pltpu.sync_copy(src_hbm.at[c, s], src_v)
    pltpu.sync_copy(dst_hbm.at[s], dst_v)

    def zrow(r, _):
        for l in range(DH // 16):
            buf0[r, pl.ds(l * 16, 16)] = jnp.zeros((16,), jnp.float32)
        return 0
    lax.fori_loop(0, B, zrow, 0)
    base = s * ROWS_PER_TILE
    chunks = [B] * (ROWS_PER_TILE // B) + (
        [ROWS_PER_TILE % B] if ROWS_PER_TILE % B else [])
    for k, n in enumerate(chunks):
        pltpu.sync_copy(buf0.at[pl.ds(0, n)], acc.at[pl.ds(base + k * B, n)])
    plsc.subcore_barrier()

    def zero_hist(nwords):
        def zh(i, _):
            hist[0, pl.ds(i * 16, 16)] = z16
            return 0
        lax.fori_loop(0, nwords // 16, zh, 0)

    def prefix_hist(nwords):
        def pb(i, carry):
            vv = hist[0, pl.ds(i * 16, 16)]
            inc = plsc.cumsum(vv)
            hist[0, pl.ds(i * 16, 16)] = inc - vv + carry
            return carry + jnp.sum(vv)
        lax.fori_loop(0, nwords // 16, pb, i32(0))

    def gen(j, m):
        for k in range(B // 16):
            ids = idb[0, pl.ds(j * B + k * 16, 16)]
            ssrc[m, pl.ds(k * 16, 16)] = plsc.load_gather(src_v, [z16, ids])
            sdst[m, pl.ds(k * 16, 16)] = plsc.load_gather(dst_v, [z16, ids])

    def gath(m, buf, sem):
        return pltpu.make_async_copy(xcat.at[ssrc.at[m]], buf, sem)

    def scat(m, buf, sem):
        return pltpu.make_async_copy(buf, acc.at[sdst.at[m]], sem)

    for hh in range(2):
        base_e = hh * HEDGE

        zero_hist(1024)

        def h1(cc, _):
            v = src_v[0, pl.ds(base_e + cc * 16, 16)]
            cell = (((v >> 2) & 63) << 4) + iota16
            cur = plsc.load_gather(hist, [z16, cell])
            plsc.store_scatter(hist, [z16, cell], cur + ones16)
            return 0
        lax.fori_loop(0, HEDGE // 16, h1, 0)
        prefix_hist(1024)

        def p1(cc, _):
            v = src_v[0, pl.ds(base_e + cc * 16, 16)]
            cell = (((v >> 2) & 63) << 4) + iota16
            cur = plsc.load_gather(hist, [z16, cell])
            plsc.store_scatter(hist, [z16, cell], cur + ones16)
            plsc.store_scatter(ida, [z16, cur], base_e + cc * 16 + iota16)
            return 0
        lax.fori_loop(0, HEDGE // 16, p1, 0)

        zero_hist(2048)

        def h2b(cc, _):
            ids = ida[0, pl.ds(cc * 16, 16)]
            v = plsc.load_gather(src_v, [z16, ids])
            cell = ((v >> 8) << 4) + iota16
            cur = plsc.load_gather(hist, [z16, cell])
            plsc.store_scatter(hist, [z16, cell], cur + ones16)
            return 0
        lax.fori_loop(0, HEDGE // 16, h2b, 0)
        prefix_hist(2048)

        def p2(cc, _):
            ids = ida[0, pl.ds(cc * 16, 16)]
            v = plsc.load_gather(src_v, [z16, ids])
            cell = ((v >> 8) << 4) + iota16
            cur = plsc.load_gather(hist, [z16, cell])
            plsc.store_scatter(hist, [z16, cell], cur + ones16)
            plsc.store_scatter(idb, [z16, cur], ids)
            return 0
        lax.fori_loop(0, HEDGE // 16, p2, 0)

        gen(0, 0)
        gath(0, buf0, g0).start()
        gen(1, 1)
        gath(1, buf1, g1).start()

        def body(q, _):
            j0 = 2 * q
            j1 = j0 + 1

            @pl.when(q >= 1)
            def _():
                scat(0, buf0, s0).wait()
                gen(j0, 0)
                gath(0, buf0, g0).start()
                gath(1, buf1, g1).wait()
                scat(1, buf1, s1).start(add=True)

                scat(1, buf1, s1).wait()
                gen(j1, 1)
                gath(1, buf1, g1).start()

            gath(0, buf0, g0).wait()
            scat(0, buf0, s0).start(add=True)
            return 0
        lax.fori_loop(0, HBLK // 2, body, 0)
        gath(1, buf1, g1).wait()
        scat(1, buf1, s1).start(add=True)
        scat(0, buf0, s0).wait()
        scat(1, buf1, s1).wait()

    plsc.subcore_barrier()

    for k, n in enumerate(chunks):
        pltpu.sync_copy(acc.at[pl.ds(base + k * B, n)], buf0.at[pl.ds(0, n)])
        pltpu.sync_copy(buf0.at[pl.ds(0, n)], h2.at[c, pl.ds(base + k * B, n)])


@jax.jit
def _sc_segment_sum(xcat, src_idx, dst_idx):
    mesh = plsc.VectorSubcoreMesh(core_axis_name="c", subcore_axis_name="s")
    return pl.kernel(
        _sc_body,
        out_type=jax.ShapeDtypeStruct((NC, ACC_ROWS, DH), jnp.float32),
        mesh=mesh,
        compiler_params=pltpu.CompilerParams(needs_layout_passes=False),
        scratch_types=[
            pltpu.VMEM((1, EPT), jnp.int32),
            pltpu.VMEM((1, EPT), jnp.int32),
            pltpu.VMEM((1, HEDGE), jnp.int32),
            pltpu.VMEM((1, HEDGE), jnp.int32),
            pltpu.VMEM((1, 2048), jnp.int32),
            pltpu.VMEM((B, DH), jnp.float32),
            pltpu.VMEM((B, DH), jnp.float32),
            pltpu.VMEM((2, B), jnp.int32),
            pltpu.VMEM((2, B), jnp.int32),
            pltpu.VMEM_SHARED((ACC_ROWS, DH), jnp.float32),
            pltpu.SemaphoreType.DMA,
            pltpu.SemaphoreType.DMA,
            pltpu.SemaphoreType.DMA,
            pltpu.SemaphoreType.DMA,
        ],
    )(xcat, src_idx, dst_idx)


def _tc_linear_body(h_ref, wt_ref, b_ref, out_ref):
    h0 = h_ref[0]
    h1 = h_ref[1]
    out_ref[...] = (
        jnp.dot(h0, wt_ref[:DH, :], preferred_element_type=jnp.float32)
        + jnp.dot(h1, wt_ref[DH:, :], preferred_element_type=jnp.float32)
        + b_ref[...]
    )


@jax.jit
def _tc_linear(h2, wt, b2):
    bn = 400
    grid = (N_NODES // bn,)
    return pl.pallas_call(
        _tc_linear_body,
        grid=grid,
        in_specs=[
            pl.BlockSpec((NC, bn, DH), lambda i: (0, i, 0)),
            pl.BlockSpec((D, D), lambda i: (0, 0)),
            pl.BlockSpec((1, D), lambda i: (0, 0)),
        ],
        out_specs=pl.BlockSpec((bn, D), lambda i: (i, 0)),
        out_shape=jax.ShapeDtypeStruct((N_NODES, D), jnp.float32),
    )(h2, wt, b2)


def kernel(x, edge_index, W, b):
    src = edge_index[0].astype(jnp.int32)
    dst = edge_index[1].astype(jnp.int32)
    e = src.shape[0]
    pad = E_PAD - e
    srcp = jnp.concatenate([src, jnp.zeros((pad,), jnp.int32)])
    dstp = jnp.concatenate([dst, jnp.full((pad,), DUMMY, jnp.int32)])
    src_idx = jnp.stack([srcp, srcp + N_NODES]).reshape(NC, NS, 1, EPT)
    dst_idx = dstp.reshape(NS, 1, EPT)
    xcat = x.reshape(N_NODES, NC, DH).transpose(1, 0, 2).reshape(
        NC * N_NODES, DH)
    h2 = _sc_segment_sum(xcat, src_idx, dst_idx)
    return _tc_linear(h2, W.T, b.reshape(1, D))


# speedup vs baseline: 1.1439x; 1.1439x over previous
import jax
import jax.numpy as jnp
from jax import lax
from jax.experimental import pallas as pl
from jax.experimental.pallas import tpu as pltpu
from jax.experimental.pallas import tpu_sc as plsc
import functools

N_NODES = 10000
D = 256
DH = 128
NC = 2
NS = 16
B = 128
NB = 80
NH = 40
E_PAD = NS * NB * B
ACC_ROWS = 10112
ROWS_PER_TILE = ACC_ROWS // NS
DUMMY = N_NODES


def _sc_body(xcat, src_hbm, dst_hbm, h2, src_v, dst_v, buf0, buf1, acc,
             sem0, sem1, ssem0, ssem1):
    c = lax.axis_index("c")
    s = lax.axis_index("s")

    def zrow(r, _):
        for l in range(DH // 16):
            buf0[r, pl.ds(l * 16, 16)] = jnp.zeros((16,), jnp.float32)
        return 0
    lax.fori_loop(0, B, zrow, 0)
    base = s * ROWS_PER_TILE
    chunks = [B] * (ROWS_PER_TILE // B) + (
        [ROWS_PER_TILE % B] if ROWS_PER_TILE % B else [])
    for k, n in enumerate(chunks):
        pltpu.sync_copy(buf0.at[pl.ds(0, n)], acc.at[pl.ds(base + k * B, n)])
    plsc.subcore_barrier()

    def gath(j, buf, sem):
        return pltpu.make_async_copy(xcat.at[src_v.at[j]], buf, sem)

    def scat(j, buf, sem):
        return pltpu.make_async_copy(buf, acc.at[dst_v.at[j]], sem)

    for h in range(NB // NH):
        pltpu.sync_copy(src_hbm.at[c, s, pl.ds(h * NH, NH)], src_v)
        pltpu.sync_copy(dst_hbm.at[s, pl.ds(h * NH, NH)], dst_v)

        gath(0, buf0, sem0).start()
        gath(1, buf1, sem1).start()

        def pair(q, _):
            j0 = 2 * q
            j1 = j0 + 1

            @pl.when(q >= 1)
            def _():
                scat(j0 - 2, buf0, ssem0).wait()
                gath(j0, buf0, sem0).start()
                gath(j0 - 1, buf1, sem1).wait()
                scat(j0 - 1, buf1, ssem1).start(add=True)

                scat(j1 - 2, buf1, ssem1).wait()
                gath(j1, buf1, sem1).start()

            gath(j0, buf0, sem0).wait()
            scat(j0, buf0, ssem0).start(add=True)
            return 0
        lax.fori_loop(0, NH // 2, pair, 0)
        gath(NH - 1, buf1, sem1).wait()
        scat(NH - 1, buf1, ssem1).start(add=True)
        scat(NH - 2, buf0, ssem0).wait()
        scat(NH - 1, buf1, ssem1).wait()
    plsc.subcore_barrier()

    pltpu.sync_copy(acc.at[pl.ds(base, ROWS_PER_TILE)],
                    h2.at[c, pl.ds(base, ROWS_PER_TILE)])


@jax.jit
def _sc_segment_sum(xcat, src_idx, dst_idx):
    mesh = plsc.VectorSubcoreMesh(core_axis_name="c", subcore_axis_name="s")
    return pl.kernel(
        _sc_body,
        out_type=jax.ShapeDtypeStruct((NC, ACC_ROWS, DH), jnp.float32),
        mesh=mesh,
        scratch_types=[
            pltpu.VMEM((NH, B), jnp.int32),
            pltpu.VMEM((NH, B), jnp.int32),
            pltpu.VMEM((B, DH), jnp.float32),
            pltpu.VMEM((B, DH), jnp.float32),
            pltpu.VMEM_SHARED((ACC_ROWS, DH), jnp.float32),
            pltpu.SemaphoreType.DMA,
            pltpu.SemaphoreType.DMA,
            pltpu.SemaphoreType.DMA,
            pltpu.SemaphoreType.DMA,
        ],
    )(xcat, src_idx, dst_idx)


def _tc_linear_body(h_ref, wt_ref, b_ref, out_ref):
    h0 = h_ref[0]
    h1 = h_ref[1]
    out_ref[...] = (
        jnp.dot(h0, wt_ref[:DH, :], preferred_element_type=jnp.float32)
        + jnp.dot(h1, wt_ref[DH:, :], preferred_element_type=jnp.float32)
        + b_ref[...]
    )


@jax.jit
def _tc_linear(h2, wt, b2):
    bn = 400
    grid = (N_NODES // bn,)
    return pl.pallas_call(
        _tc_linear_body,
        grid=grid,
        in_specs=[
            pl.BlockSpec((NC, bn, DH), lambda i: (0, i, 0)),
            pl.BlockSpec((D, D), lambda i: (0, 0)),
            pl.BlockSpec((1, D), lambda i: (0, 0)),
        ],
        out_specs=pl.BlockSpec((bn, D), lambda i: (i, 0)),
        out_shape=jax.ShapeDtypeStruct((N_NODES, D), jnp.float32),
    )(h2, wt, b2)


def kernel(x, edge_index, W, b):
    src = edge_index[0].astype(jnp.int32)
    dst = edge_index[1].astype(jnp.int32)
    e = src.shape[0]
    pad = E_PAD - e
    srcp = jnp.concatenate([src, jnp.zeros((pad,), jnp.int32)])
    dstp = jnp.concatenate([dst, jnp.full((pad,), DUMMY, jnp.int32)])
    src_idx = jnp.stack([srcp, srcp + N_NODES]).reshape(NC, NS, NB, B)
    dst_idx = dstp.reshape(NS, NB, B)
    xcat = x.reshape(N_NODES, NC, DH).transpose(1, 0, 2).reshape(
        NC * N_NODES, DH)
    h2 = _sc_segment_sum(xcat, src_idx, dst_idx)
    return _tc_linear(h2, W.T, b.reshape(1, D))


# speedup vs baseline: 1.1456x; 1.0014x over previous
import jax
import jax.numpy as jnp
from jax import lax
from jax.experimental import pallas as pl
from jax.experimental.pallas import tpu as pltpu
from jax.experimental.pallas import tpu_sc as plsc

N_NODES = 10000
D = 256
DH = 128
NC = 2
NS = 16
B = 128
NB = 80
NH = 40
E_PAD = NS * NB * B
ACC_ROWS = 10112
ROWS_PER_TILE = ACC_ROWS // NS
DUMMY = N_NODES


def _sc_body(xcat, src_hbm, dst_hbm, h2, src_v, dst_v, buf0, buf1, acc,
             sem0, sem1, ssem0, ssem1):
    c = lax.axis_index("c")
    s = lax.axis_index("s")

    def zrow(r, _):
        for l in range(DH // 16):
            buf0[r, pl.ds(l * 16, 16)] = jnp.zeros((16,), jnp.float32)
        return 0
    lax.fori_loop(0, B, zrow, 0)
    base = s * ROWS_PER_TILE
    chunks = [B] * (ROWS_PER_TILE // B) + (
        [ROWS_PER_TILE % B] if ROWS_PER_TILE % B else [])
    for k, n in enumerate(chunks):
        pltpu.sync_copy(buf0.at[pl.ds(0, n)], acc.at[pl.ds(base + k * B, n)])
    plsc.subcore_barrier()

    def gath(j, buf, sem):
        return pltpu.make_async_copy(xcat.at[src_v.at[j]], buf, sem)

    def scat(j, buf, sem):
        return pltpu.make_async_copy(buf, acc.at[dst_v.at[j]], sem)

    for h in range(NB // NH):
        pltpu.sync_copy(src_hbm.at[c, s, pl.ds(h * NH, NH)], src_v)
        pltpu.sync_copy(dst_hbm.at[s, pl.ds(h * NH, NH)], dst_v)

        gath(0, buf0, sem0).start()
        gath(1, buf1, sem1).start()

        def pair(q, _):
            j0 = 2 * q
            j1 = j0 + 1

            @pl.when(q >= 1)
            def _():
                scat(j0 - 2, buf0, ssem0).wait()
                gath(j0, buf0, sem0).start()
                gath(j0 - 1, buf1, sem1).wait()
                scat(j0 - 1, buf1, ssem1).start(add=True)

                scat(j1 - 2, buf1, ssem1).wait()
                gath(j1, buf1, sem1).start()

            gath(j0, buf0, sem0).wait()
            scat(j0, buf0, ssem0).start(add=True)
            return 0
        lax.fori_loop(0, NH // 2, pair, 0)
        gath(NH - 1, buf1, sem1).wait()
        scat(NH - 1, buf1, ssem1).start(add=True)
        scat(NH - 2, buf0, ssem0).wait()
        scat(NH - 1, buf1, ssem1).wait()
    plsc.subcore_barrier()

    pltpu.sync_copy(acc.at[pl.ds(base, ROWS_PER_TILE)],
                    h2.at[c, pl.ds(base, ROWS_PER_TILE)])


@jax.jit
def _sc_segment_sum(xcat, src_idx, dst_idx):
    mesh = plsc.VectorSubcoreMesh(core_axis_name="c", subcore_axis_name="s")
    return pl.kernel(
        _sc_body,
        out_type=jax.ShapeDtypeStruct((NC, ACC_ROWS, DH), jnp.float32),
        mesh=mesh,
        scratch_types=[
            pltpu.VMEM((NH, B), jnp.int32),
            pltpu.VMEM((NH, B), jnp.int32),
            pltpu.VMEM((B, DH), jnp.float32),
            pltpu.VMEM((B, DH), jnp.float32),
            pltpu.VMEM_SHARED((ACC_ROWS, DH), jnp.float32),
            pltpu.SemaphoreType.DMA,
            pltpu.SemaphoreType.DMA,
            pltpu.SemaphoreType.DMA,
            pltpu.SemaphoreType.DMA,
        ],
    )(xcat, src_idx, dst_idx)


def _tc_linear_body(h_ref, wt_ref, b_ref, out_ref):
    h0 = h_ref[0]
    h1 = h_ref[1]
    out_ref[...] = (
        jnp.dot(h0, wt_ref[:DH, :], preferred_element_type=jnp.float32)
        + jnp.dot(h1, wt_ref[DH:, :], preferred_element_type=jnp.float32)
        + b_ref[...]
    )


@jax.jit
def _tc_linear(h2, wt, b2):
    bn = 400
    grid = (N_NODES // bn,)
    return pl.pallas_call(
        _tc_linear_body,
        grid=grid,
        in_specs=[
            pl.BlockSpec((NC, bn, DH), lambda i: (0, i, 0)),
            pl.BlockSpec((D, D), lambda i: (0, 0)),
            pl.BlockSpec((1, D), lambda i: (0, 0)),
        ],
        out_specs=pl.BlockSpec((bn, D), lambda i: (i, 0)),
        out_shape=jax.ShapeDtypeStruct((N_NODES, D), jnp.float32),
    )(h2, wt, b2)


def kernel(x, edge_index, W, b):
    src = edge_index[0].astype(jnp.int32)
    dst = edge_index[1].astype(jnp.int32)
    e = src.shape[0]
    pad = E_PAD - e
    srcp = jnp.concatenate([src, jnp.zeros((pad,), jnp.int32)])
    dstp = jnp.concatenate([dst, jnp.full((pad,), DUMMY, jnp.int32)])
    src_idx = jnp.stack([srcp, srcp + N_NODES]).reshape(NC, NS, NB, B)
    dst_idx = dstp.reshape(NS, NB, B)
    xcat = x.reshape(N_NODES, NC, DH).transpose(1, 0, 2).reshape(
        NC * N_NODES, DH)
    h2 = _sc_segment_sum(xcat, src_idx, dst_idx)
    return _tc_linear(h2, W.T, b.reshape(1, D))
